# baseline (device time: 32181 ns/iter reference)
import jax
import jax.numpy as jnp
from jax import lax
from jax.experimental import pallas as pl
from jax.experimental.pallas import tpu as pltpu

N_DEV = 32
B, SQ, D = 2, 128, 512
HQ_LOC = 8
KV_LOC = 2
GRP = 4
DH = 64
R = B * SQ
CH = R // N_DEV
BF = jnp.bfloat16


def kernel(x, Wq, Wo, Wk, Wv):
    x2 = x.reshape(R, D)

    def body(x_hbm, wq_hbm, wo_hbm, wk_hbm, wv_hbm, out_ref,
             x_v, wq_v, wo_v, kv_v, part_ref, recv_ref, red_ref, stage_ref,
             ld_sems, rs_send, rs_recv, ag_send, ag_recv, loc_sem):
        my = lax.axis_index("i")

        barrier = pltpu.get_barrier_semaphore()
        for o in range(1, N_DEV):
            pl.semaphore_signal(
                barrier, inc=1,
                device_id=((my + o) % N_DEV,),
                device_id_type=pl.DeviceIdType.MESH,
            )

        ld_x = pltpu.make_async_copy(x_hbm, x_v, ld_sems.at[0])
        ld_wq = pltpu.make_async_copy(wq_hbm, wq_v, ld_sems.at[1])
        ld_wk = pltpu.make_async_copy(
            wk_hbm.at[:, pl.ds(my * KV_LOC * DH, KV_LOC * DH)],
            kv_v.at[0], ld_sems.at[2])
        ld_wv = pltpu.make_async_copy(
            wv_hbm.at[:, pl.ds(my * KV_LOC * DH, KV_LOC * DH)],
            kv_v.at[1], ld_sems.at[3])
        ld_wo = pltpu.make_async_copy(wo_hbm, wo_v, ld_sems.at[4])
        for ld in (ld_x, ld_wq, ld_wk, ld_wv, ld_wo):
            ld.start()

        ld_x.wait()
        xb = x_v[...].astype(BF)
        ld_wq.wait()
        qb = lax.dot_general(
            xb, wq_v[...].astype(BF), (((1,), (0,)), ((), ())),
            preferred_element_type=jnp.float32).astype(BF)
        ld_wk.wait()
        kb = lax.dot_general(
            xb, kv_v[0].astype(BF), (((1,), (0,)), ((), ())),
            preferred_element_type=jnp.float32).astype(BF)
        ld_wv.wait()
        vb = lax.dot_general(
            xb, kv_v[1].astype(BF), (((1,), (0,)), ((), ())),
            preferred_element_type=jnp.float32).astype(BF)

        o_blocks = {}
        for b in range(B):
            r0 = b * SQ
            for g in range(KV_LOC):
                qstack = jnp.concatenate(
                    [qb[r0:r0 + SQ, (GRP * g + r) * DH:(GRP * g + r + 1) * DH]
                     for r in range(GRP)], axis=0)
                kbg = kb[r0:r0 + SQ, g * DH:(g + 1) * DH]
                vbg = vb[r0:r0 + SQ, g * DH:(g + 1) * DH]
                s = lax.dot_general(
                    qstack, kbg, (((1,), (1,)), ((), ())),
                    preferred_element_type=jnp.float32) * 0.125
                m = jnp.max(s, axis=1, keepdims=True)
                p = jnp.exp(s - m)
                l = jnp.sum(p, axis=1, keepdims=True)
                o = lax.dot_general(
                    p.astype(BF), vbg, (((1,), (0,)), ((), ())),
                    preferred_element_type=jnp.float32) / l
                o_blocks[(b, g)] = o

        rows = []
        for b in range(B):
            cols = []
            for h in range(HQ_LOC):
                g, r = h // GRP, h % GRP
                cols.append(o_blocks[(b, g)][r * SQ:(r + 1) * SQ])
            rows.append(jnp.concatenate(cols, axis=1))
        attn = jnp.concatenate(rows, axis=0).astype(BF)

        ld_wo.wait()
        part_ref[...] = lax.dot_general(
            attn, wo_v[...].astype(BF), (((1,), (0,)), ((), ())),
            preferred_element_type=jnp.float32).astype(BF)

        pl.semaphore_wait(barrier, N_DEV - 1)

        loc = pltpu.make_async_copy(
            part_ref.at[pl.ds(my * CH, CH), :], recv_ref.at[my], loc_sem)
        loc.start()
        rs_rdmas = []
        for o in range(1, N_DEV):
            j = (my + o) % N_DEV
            rdma = pltpu.make_async_remote_copy(
                src_ref=part_ref.at[pl.ds(j * CH, CH), :],
                dst_ref=recv_ref.at[my],
                send_sem=rs_send.at[o],
                recv_sem=rs_recv.at[my],
                device_id=(j,),
                device_id_type=pl.DeviceIdType.MESH,
            )
            rdma.start()
            rs_rdmas.append(rdma)
        loc.wait()
        for o in range(1, N_DEV):
            s = (my + o) % N_DEV
            pltpu.make_async_remote_copy(
                src_ref=part_ref.at[pl.ds(s * CH, CH), :],
                dst_ref=recv_ref.at[s],
                send_sem=rs_send.at[o],
                recv_sem=rs_recv.at[s],
                device_id=(s,),
                device_id_type=pl.DeviceIdType.MESH,
            ).wait_recv()

        vals = [recv_ref[j].astype(jnp.float32) for j in range(N_DEV)]
        while len(vals) > 1:
            vals = [vals[k] + vals[k + 1] for k in range(0, len(vals), 2)]
        red_ref[...] = vals[0].astype(BF)

        loc2 = pltpu.make_async_copy(red_ref, stage_ref.at[my], loc_sem)
        loc2.start()
        ag_rdmas = []
        for o in range(1, N_DEV):
            j = (my + o) % N_DEV
            rdma = pltpu.make_async_remote_copy(
                src_ref=red_ref,
                dst_ref=stage_ref.at[my],
                send_sem=ag_send.at[o],
                recv_sem=ag_recv.at[my],
                device_id=(j,),
                device_id_type=pl.DeviceIdType.MESH,
            )
            rdma.start()
            ag_rdmas.append(rdma)
        loc2.wait()
        for o in range(1, N_DEV):
            s = (my + o) % N_DEV
            pltpu.make_async_remote_copy(
                src_ref=red_ref,
                dst_ref=stage_ref.at[s],
                send_sem=ag_send.at[o],
                recv_sem=ag_recv.at[s],
                device_id=(s,),
                device_id_type=pl.DeviceIdType.MESH,
            ).wait_recv()

        out_ref[...] = stage_ref[...].reshape(R, D).astype(jnp.float32)

        for rdma in rs_rdmas:
            rdma.wait_send()
        for rdma in ag_rdmas:
            rdma.wait_send()

    out = pl.pallas_call(
        body,
        out_shape=jax.ShapeDtypeStruct((R, D), jnp.float32),
        in_specs=[pl.BlockSpec(memory_space=pl.ANY)] * 5,
        out_specs=pl.BlockSpec(memory_space=pltpu.VMEM),
        scratch_shapes=[
            pltpu.VMEM((R, D), jnp.float32),
            pltpu.VMEM((D, D), jnp.float32),
            pltpu.VMEM((D, D), jnp.float32),
            pltpu.VMEM((2, D, KV_LOC * DH), jnp.float32),
            pltpu.VMEM((R, D), BF),
            pltpu.VMEM((N_DEV, CH, D), BF),
            pltpu.VMEM((CH, D), BF),
            pltpu.VMEM((N_DEV, CH, D), BF),
            pltpu.SemaphoreType.DMA((5,)),
            pltpu.SemaphoreType.DMA((N_DEV,)),
            pltpu.SemaphoreType.DMA((N_DEV,)),
            pltpu.SemaphoreType.DMA((N_DEV,)),
            pltpu.SemaphoreType.DMA((N_DEV,)),
            pltpu.SemaphoreType.DMA,
        ],
        compiler_params=pltpu.CompilerParams(collective_id=0),
    )(x2, Wq, Wo, Wk, Wv)
    return out.reshape(B, SQ, D)


# device time: 26820 ns/iter; 1.1999x vs baseline; 1.1999x over previous
import jax
import jax.numpy as jnp
from jax import lax
from jax.experimental import pallas as pl
from jax.experimental.pallas import tpu as pltpu

N_DEV = 32
B, SQ, D = 2, 128, 512
HQ_LOC = 8
KV_LOC = 2
GRP = 4
DH = 64
R = B * SQ
CH = R // N_DEV
BF = jnp.bfloat16


def kernel(x, Wq, Wo, Wk, Wv):
    i = lax.axis_index("i")
    x2 = x.reshape(R, D)
    wk_loc = lax.dynamic_slice(Wk, (0, i * KV_LOC * DH), (D, KV_LOC * DH))
    wv_loc = lax.dynamic_slice(Wv, (0, i * KV_LOC * DH), (D, KV_LOC * DH))

    def body(x_hbm, wq_hbm, wo_hbm, wk_ref, wv_ref, out_ref,
             x_v, wq_v, wo_v, part_ref, recv_ref, red_ref, stage_ref,
             ld_sems, rs_send, rs_recv, ag_send, ag_recv, loc_sem):
        my = lax.axis_index("i")

        barrier = pltpu.get_barrier_semaphore()
        for o in range(1, N_DEV):
            pl.semaphore_signal(
                barrier, inc=1,
                device_id=((my + o) % N_DEV,),
                device_id_type=pl.DeviceIdType.MESH,
            )

        ld_x = pltpu.make_async_copy(x_hbm, x_v, ld_sems.at[0])
        ld_wq = pltpu.make_async_copy(wq_hbm, wq_v, ld_sems.at[1])
        ld_wo = pltpu.make_async_copy(wo_hbm, wo_v, ld_sems.at[2])
        for ld in (ld_x, ld_wq, ld_wo):
            ld.start()

        ld_x.wait()
        xb = x_v[...].astype(BF)
        ld_wq.wait()
        qb = lax.dot_general(
            xb, wq_v[...].astype(BF), (((1,), (0,)), ((), ())),
            preferred_element_type=jnp.float32).astype(BF)
        kb = lax.dot_general(
            xb, wk_ref[...].astype(BF), (((1,), (0,)), ((), ())),
            preferred_element_type=jnp.float32).astype(BF)
        vb = lax.dot_general(
            xb, wv_ref[...].astype(BF), (((1,), (0,)), ((), ())),
            preferred_element_type=jnp.float32).astype(BF)

        o_blocks = {}
        for b in range(B):
            r0 = b * SQ
            for g in range(KV_LOC):
                qstack = jnp.concatenate(
                    [qb[r0:r0 + SQ, (GRP * g + r) * DH:(GRP * g + r + 1) * DH]
                     for r in range(GRP)], axis=0)
                kbg = kb[r0:r0 + SQ, g * DH:(g + 1) * DH]
                vbg = vb[r0:r0 + SQ, g * DH:(g + 1) * DH]
                s = lax.dot_general(
                    qstack, kbg, (((1,), (1,)), ((), ())),
                    preferred_element_type=jnp.float32) * 0.125
                m = jnp.max(s, axis=1, keepdims=True)
                p = jnp.exp(s - m)
                l = jnp.sum(p, axis=1, keepdims=True)
                o = lax.dot_general(
                    p.astype(BF), vbg, (((1,), (0,)), ((), ())),
                    preferred_element_type=jnp.float32) / l
                o_blocks[(b, g)] = o

        rows = []
        for b in range(B):
            cols = []
            for h in range(HQ_LOC):
                g, r = h // GRP, h % GRP
                cols.append(o_blocks[(b, g)][r * SQ:(r + 1) * SQ])
            rows.append(jnp.concatenate(cols, axis=1))
        attn = jnp.concatenate(rows, axis=0).astype(BF)

        ld_wo.wait()
        part_ref[...] = lax.dot_general(
            attn, wo_v[...].astype(BF), (((1,), (0,)), ((), ())),
            preferred_element_type=jnp.float32).astype(BF)

        pl.semaphore_wait(barrier, N_DEV - 1)

        loc = pltpu.make_async_copy(
            part_ref.at[pl.ds(my * CH, CH), :], recv_ref.at[my], loc_sem)
        loc.start()
        rs_rdmas = []
        for o in range(1, N_DEV):
            j = (my + o) % N_DEV
            rdma = pltpu.make_async_remote_copy(
                src_ref=part_ref.at[pl.ds(j * CH, CH), :],
                dst_ref=recv_ref.at[my],
                send_sem=rs_send.at[o],
                recv_sem=rs_recv.at[my],
                device_id=(j,),
                device_id_type=pl.DeviceIdType.MESH,
            )
            rdma.start()
            rs_rdmas.append(rdma)
        loc.wait()
        for o in range(1, N_DEV):
            s = (my + o) % N_DEV
            pltpu.make_async_remote_copy(
                src_ref=part_ref.at[pl.ds(s * CH, CH), :],
                dst_ref=recv_ref.at[s],
                send_sem=rs_send.at[o],
                recv_sem=rs_recv.at[s],
                device_id=(s,),
                device_id_type=pl.DeviceIdType.MESH,
            ).wait_recv()

        vals = [recv_ref[j].astype(jnp.float32) for j in range(N_DEV)]
        while len(vals) > 1:
            vals = [vals[k] + vals[k + 1] for k in range(0, len(vals), 2)]
        red_ref[...] = vals[0].astype(BF)

        loc2 = pltpu.make_async_copy(red_ref, stage_ref.at[my], loc_sem)
        loc2.start()
        ag_rdmas = []
        for o in range(1, N_DEV):
            j = (my + o) % N_DEV
            rdma = pltpu.make_async_remote_copy(
                src_ref=red_ref,
                dst_ref=stage_ref.at[my],
                send_sem=ag_send.at[o],
                recv_sem=ag_recv.at[my],
                device_id=(j,),
                device_id_type=pl.DeviceIdType.MESH,
            )
            rdma.start()
            ag_rdmas.append(rdma)
        loc2.wait()
        for o in range(1, N_DEV):
            s = (my + o) % N_DEV
            pltpu.make_async_remote_copy(
                src_ref=red_ref,
                dst_ref=stage_ref.at[s],
                send_sem=ag_send.at[o],
                recv_sem=ag_recv.at[s],
                device_id=(s,),
                device_id_type=pl.DeviceIdType.MESH,
            ).wait_recv()

        out_ref[...] = stage_ref[...].reshape(R, D).astype(jnp.float32)

        for rdma in rs_rdmas:
            rdma.wait_send()
        for rdma in ag_rdmas:
            rdma.wait_send()

    out = pl.pallas_call(
        body,
        out_shape=jax.ShapeDtypeStruct((R, D), jnp.float32),
        in_specs=[pl.BlockSpec(memory_space=pl.ANY)] * 3
        + [pl.BlockSpec(memory_space=pltpu.VMEM)] * 2,
        out_specs=pl.BlockSpec(memory_space=pltpu.VMEM),
        scratch_shapes=[
            pltpu.VMEM((R, D), jnp.float32),
            pltpu.VMEM((D, D), jnp.float32),
            pltpu.VMEM((D, D), jnp.float32),
            pltpu.VMEM((R, D), BF),
            pltpu.VMEM((N_DEV, CH, D), BF),
            pltpu.VMEM((CH, D), BF),
            pltpu.VMEM((N_DEV, CH, D), BF),
            pltpu.SemaphoreType.DMA((3,)),
            pltpu.SemaphoreType.DMA((N_DEV,)),
            pltpu.SemaphoreType.DMA((N_DEV,)),
            pltpu.SemaphoreType.DMA((N_DEV,)),
            pltpu.SemaphoreType.DMA((N_DEV,)),
            pltpu.SemaphoreType.DMA,
        ],
        compiler_params=pltpu.CompilerParams(collective_id=0),
    )(x2, Wq, Wo, wk_loc, wv_loc)
    return out.reshape(B, SQ, D)


# device time: 24041 ns/iter; 1.3386x vs baseline; 1.1156x over previous
import jax
import jax.numpy as jnp
from jax import lax
from jax.experimental import pallas as pl
from jax.experimental.pallas import tpu as pltpu

N_DEV = 32
B, SQ, D = 2, 128, 512
HQ_LOC = 8
KV_LOC = 2
GRP = 4
DH = 64
R = B * SQ
CH = R // N_DEV
BF = jnp.bfloat16


def kernel(x, Wq, Wo, Wk, Wv):
    i = lax.axis_index("i")
    x2 = x.reshape(R, D)
    wk_loc = lax.dynamic_slice(Wk, (0, i * KV_LOC * DH), (D, KV_LOC * DH))
    wv_loc = lax.dynamic_slice(Wv, (0, i * KV_LOC * DH), (D, KV_LOC * DH))

    def body(x_hbm, wq_hbm, wo_hbm, wk_ref, wv_ref, out_ref,
             x_v, wq_v, wo_v, part_ref, recv_ref, red_ref, stage_ref,
             ld_sems, rs_send, rs_recv, ag_send, ag_recv, loc_sem):
        my = lax.axis_index("i")

        barrier = pltpu.get_barrier_semaphore()
        for o in range(1, N_DEV):
            pl.semaphore_signal(
                barrier, inc=1,
                device_id=((my + o) % N_DEV,),
                device_id_type=pl.DeviceIdType.MESH,
            )

        ld_x = pltpu.make_async_copy(x_hbm, x_v, ld_sems.at[0])
        ld_wq = pltpu.make_async_copy(wq_hbm, wq_v, ld_sems.at[1])
        ld_wo = pltpu.make_async_copy(wo_hbm, wo_v, ld_sems.at[2])
        for ld in (ld_x, ld_wq, ld_wo):
            ld.start()

        ld_x.wait()
        xb = x_v[...].astype(BF)
        ld_wq.wait()
        qb = lax.dot_general(
            xb, wq_v[...].astype(BF), (((1,), (0,)), ((), ())),
            preferred_element_type=jnp.float32).astype(BF)
        kb = lax.dot_general(
            xb, wk_ref[...].astype(BF), (((1,), (0,)), ((), ())),
            preferred_element_type=jnp.float32).astype(BF)
        vb = lax.dot_general(
            xb, wv_ref[...].astype(BF), (((1,), (0,)), ((), ())),
            preferred_element_type=jnp.float32).astype(BF)

        o_blocks = {}
        for b in range(B):
            r0 = b * SQ
            for g in range(KV_LOC):
                qstack = jnp.concatenate(
                    [qb[r0:r0 + SQ, (GRP * g + r) * DH:(GRP * g + r + 1) * DH]
                     for r in range(GRP)], axis=0)
                kbg = kb[r0:r0 + SQ, g * DH:(g + 1) * DH]
                vbg = vb[r0:r0 + SQ, g * DH:(g + 1) * DH]
                s = lax.dot_general(
                    qstack, kbg, (((1,), (1,)), ((), ())),
                    preferred_element_type=jnp.float32) * 0.125
                m = jnp.max(s, axis=1, keepdims=True)
                p = jnp.exp(s - m)
                l = jnp.sum(p, axis=1, keepdims=True)
                o = lax.dot_general(
                    p.astype(BF), vbg, (((1,), (0,)), ((), ())),
                    preferred_element_type=jnp.float32) / l
                o_blocks[(b, g)] = o

        rows = []
        for b in range(B):
            cols = []
            for h in range(HQ_LOC):
                g, r = h // GRP, h % GRP
                cols.append(o_blocks[(b, g)][r * SQ:(r + 1) * SQ])
            rows.append(jnp.concatenate(cols, axis=1))
        attn = jnp.concatenate(rows, axis=0).astype(BF)

        ld_wo.wait()
        part_ref[...] = lax.dot_general(
            attn, wo_v[...].astype(BF), (((1,), (0,)), ((), ())),
            preferred_element_type=jnp.float32).astype(BF)

        pl.semaphore_wait(barrier, N_DEV - 1)

        loc = pltpu.make_async_copy(
            part_ref.at[pl.ds(my * CH, CH), :], recv_ref.at[my], loc_sem)
        loc.start()
        rs_rdmas = []
        for o in range(1, N_DEV):
            j = (my + o) % N_DEV
            rdma = pltpu.make_async_remote_copy(
                src_ref=part_ref.at[pl.ds(j * CH, CH), :],
                dst_ref=recv_ref.at[my],
                send_sem=rs_send.at[o],
                recv_sem=rs_recv.at[my],
                device_id=(j,),
                device_id_type=pl.DeviceIdType.MESH,
            )
            rdma.start()
            rs_rdmas.append(rdma)
        loc.wait()
        for o in range(1, N_DEV):
            s = (my + o) % N_DEV
            pltpu.make_async_remote_copy(
                src_ref=part_ref.at[pl.ds(s * CH, CH), :],
                dst_ref=recv_ref.at[s],
                send_sem=rs_send.at[o],
                recv_sem=rs_recv.at[s],
                device_id=(s,),
                device_id_type=pl.DeviceIdType.MESH,
            ).wait_recv()

        vals = [recv_ref[j].astype(jnp.float32) for j in range(N_DEV)]
        while len(vals) > 1:
            vals = [vals[k] + vals[k + 1] for k in range(0, len(vals), 2)]
        red_ref[...] = vals[0].astype(BF)

        loc2 = pltpu.make_async_copy(red_ref, stage_ref.at[my], loc_sem)
        loc2.start()
        ag_rdmas = []
        for o in range(1, N_DEV):
            j = (my + o) % N_DEV
            rdma = pltpu.make_async_remote_copy(
                src_ref=red_ref,
                dst_ref=stage_ref.at[my],
                send_sem=ag_send.at[o],
                recv_sem=ag_recv.at[my],
                device_id=(j,),
                device_id_type=pl.DeviceIdType.MESH,
            )
            rdma.start()
            ag_rdmas.append(rdma)
        loc2.wait()
        for o in range(1, N_DEV):
            s = (my + o) % N_DEV
            pltpu.make_async_remote_copy(
                src_ref=red_ref,
                dst_ref=stage_ref.at[s],
                send_sem=ag_send.at[o],
                recv_sem=ag_recv.at[s],
                device_id=(s,),
                device_id_type=pl.DeviceIdType.MESH,
            ).wait_recv()

        out_ref[...] = stage_ref[...].reshape(R, D).astype(jnp.float32)

        for rdma in rs_rdmas:
            rdma.wait_send()
        for rdma in ag_rdmas:
            rdma.wait_send()

    out = pl.pallas_call(
        body,
        out_shape=jax.ShapeDtypeStruct((R, D), jnp.float32),
        in_specs=[pl.BlockSpec(memory_space=pltpu.MemorySpace.HBM)] * 3
        + [pl.BlockSpec(memory_space=pltpu.VMEM)] * 2,
        out_specs=pl.BlockSpec(memory_space=pltpu.VMEM),
        scratch_shapes=[
            pltpu.VMEM((R, D), jnp.float32),
            pltpu.VMEM((D, D), jnp.float32),
            pltpu.VMEM((D, D), jnp.float32),
            pltpu.VMEM((R, D), BF),
            pltpu.VMEM((N_DEV, CH, D), BF),
            pltpu.VMEM((CH, D), BF),
            pltpu.VMEM((N_DEV, CH, D), BF),
            pltpu.SemaphoreType.DMA((3,)),
            pltpu.SemaphoreType.DMA((N_DEV,)),
            pltpu.SemaphoreType.DMA((N_DEV,)),
            pltpu.SemaphoreType.DMA((N_DEV,)),
            pltpu.SemaphoreType.DMA((N_DEV,)),
            pltpu.SemaphoreType.DMA,
        ],
        compiler_params=pltpu.CompilerParams(collective_id=0),
    )(x2, Wq, Wo, wk_loc, wv_loc)
    return out.reshape(B, SQ, D)
